# baseline (speedup 1.0000x reference)
import functools

import jax
import jax.numpy as jnp
from jax.experimental import pallas as pl
from jax.experimental.pallas import tpu as pltpu

_C1, _H1, _W1 = 64, 32, 32
_C2, _H2, _W2 = 128, 16, 16
_N = 128


_OH = 4


def _body(x_ref, wt_ref, b_ref, o_ref, *, oh):
    x = x_ref[...]
    cols = []
    for m in range(oh):
        a = x[:, 2 * m] + x[:, 2 * m + 1]
        for k in range(_W2):
            s = a[:, 2 * k, :] + a[:, 2 * k + 1, :]
            cols.append((s * 0.25).T)
    pt = jnp.stack(cols, axis=1)
    y = jnp.dot(pt.reshape(_N * oh * _W2, _C1), wt_ref[...],
                preferred_element_type=jnp.float32)
    o_ref[...] = (y + b_ref[...]).reshape(_N, oh, _W2, _C2)


@jax.jit
def kernel(x_nchw, weight, bias):
    n = x_nchw.shape[0]
    xt = jnp.transpose(x_nchw, (1, 2, 3, 0))
    wt = weight.astype(jnp.float32).T
    b = bias.astype(jnp.float32).reshape(1, _C2)

    out = pl.pallas_call(
        functools.partial(_body, oh=_OH),
        out_shape=jax.ShapeDtypeStruct((n, _H2, _W2, _C2), x_nchw.dtype),
        grid_spec=pltpu.PrefetchScalarGridSpec(
            num_scalar_prefetch=0,
            grid=(_H2 // _OH,),
            in_specs=[
                pl.BlockSpec((_C1, 2 * _OH, _W1, _N), lambda j: (0, j, 0, 0)),
                pl.BlockSpec((_C1, _C2), lambda j: (0, 0)),
                pl.BlockSpec((1, _C2), lambda j: (0, 0)),
            ],
            out_specs=pl.BlockSpec((n, _OH, _W2, _C2), lambda j: (0, j, 0, 0)),
        ),
        compiler_params=pltpu.CompilerParams(
            dimension_semantics=("parallel",),
            vmem_limit_bytes=64 << 20,
        ),
    )(xt, wt, b)
    return jnp.transpose(out, (0, 3, 1, 2))


# speedup vs baseline: 1.0400x; 1.0400x over previous
import functools

import jax
import jax.numpy as jnp
from jax.experimental import pallas as pl
from jax.experimental.pallas import tpu as pltpu

_C1, _H1, _W1 = 64, 32, 32
_C2, _H2, _W2 = 128, 16, 16
_N = 128


_OH = 2


def _body(x_ref, wt_ref, b_ref, o_ref, *, oh):
    x = x_ref[...]
    cols = []
    for m in range(oh):
        a = x[:, 2 * m] + x[:, 2 * m + 1]
        for k in range(_W2):
            s = a[:, 2 * k, :] + a[:, 2 * k + 1, :]
            cols.append((s * 0.25).T)
    pt = jnp.stack(cols, axis=1)
    y = jnp.dot(pt.reshape(_N * oh * _W2, _C1), wt_ref[...],
                preferred_element_type=jnp.float32)
    o_ref[...] = (y + b_ref[...]).reshape(_N, oh, _W2, _C2)


@jax.jit
def kernel(x_nchw, weight, bias):
    n = x_nchw.shape[0]
    xt = jnp.transpose(x_nchw, (1, 2, 3, 0))
    wt = weight.astype(jnp.float32).T
    b = bias.astype(jnp.float32).reshape(1, _C2)

    out = pl.pallas_call(
        functools.partial(_body, oh=_OH),
        out_shape=jax.ShapeDtypeStruct((n, _H2, _W2, _C2), x_nchw.dtype),
        grid_spec=pltpu.PrefetchScalarGridSpec(
            num_scalar_prefetch=0,
            grid=(_H2 // _OH,),
            in_specs=[
                pl.BlockSpec((_C1, 2 * _OH, _W1, _N), lambda j: (0, j, 0, 0)),
                pl.BlockSpec((_C1, _C2), lambda j: (0, 0)),
                pl.BlockSpec((1, _C2), lambda j: (0, 0)),
            ],
            out_specs=pl.BlockSpec((n, _OH, _W2, _C2), lambda j: (0, j, 0, 0)),
        ),
        compiler_params=pltpu.CompilerParams(
            dimension_semantics=("parallel",),
            vmem_limit_bytes=64 << 20,
        ),
    )(xt, wt, b)
    return jnp.transpose(out, (0, 3, 1, 2))
